# baseline (speedup 1.0000x reference)
import functools

import jax
import jax.numpy as jnp
from jax import lax
from jax.experimental import pallas as pl
from jax.experimental.pallas import tpu as pltpu
from jax.experimental.pallas import tpu_sc as plsc

_B = 64
_NSLOT = 5
_LEAD = 3
_KB = 8


def _sc_segment_sum(emb, ei3, adj2d, zeros):
    n, d = emb.shape
    _, nrows, b = ei3.shape
    assert b == _B
    info = plsc.get_sparse_core_info()
    nc, ns, lanes = info.num_cores, info.num_subcores, info.num_lanes
    nw = nc * ns
    per_w = nrows // nw
    assert per_w % _KB == 0 and nrows == per_w * nw
    g_iters = per_w // _NSLOT
    vregs = d // lanes

    rows_per_sub = (n // ns) // 8 * 8
    tail_start = ns * rows_per_sub
    tail = n - tail_start

    mesh = plsc.VectorSubcoreMesh(core_axis_name="c", subcore_axis_name="s")

    @functools.partial(
        pl.kernel,
        mesh=mesh,
        out_type=jax.ShapeDtypeStruct((nc, n, d), jnp.float32),
        compiler_params=pltpu.CompilerParams(needs_layout_passes=False),
        scratch_types=[
            pltpu.VMEM((2, _KB, b), jnp.int32),
            pltpu.VMEM((2, _KB, b), jnp.int32),
            pltpu.VMEM((2, _KB, b), jnp.float32),
            pltpu.VMEM((_NSLOT, b, d), jnp.float32),
            pltpu.VMEM_SHARED((n, d), jnp.float32),
            [pltpu.SemaphoreType.DMA] * _NSLOT,
            [pltpu.SemaphoreType.DMA] * _NSLOT,
            pltpu.SemaphoreType.DMA,
        ],
    )
    def sc_k(emb_hbm, ei_hbm, adj_hbm, zeros_hbm, part_hbm,
             src_k, dst_k, adj_k, rows, acc, g_sems, s_sems, i_sem):
        c = lax.axis_index("c")
        s = lax.axis_index("s")
        w = c * ns + s
        start = w * per_w

        def issue_idx(bi):
            off = pl.multiple_of(start + bi * _KB, 8)
            sl = bi % 2
            pltpu.async_copy(ei_hbm.at[1, pl.ds(off, _KB)], src_k.at[sl], i_sem)
            pltpu.async_copy(ei_hbm.at[0, pl.ds(off, _KB)], dst_k.at[sl], i_sem)
            pltpu.async_copy(adj_hbm.at[pl.ds(off, _KB)], adj_k.at[sl], i_sem)

        def wait_idx():
            for ref in (src_k, dst_k, adj_k):
                pltpu.make_async_copy(
                    ei_hbm.at[0, pl.ds(start, _KB)], ref.at[0], i_sem
                ).wait()

        def issue_gather(t, slot):
            pltpu.async_copy(
                emb_hbm.at[src_k.at[(t // _KB) % 2, t % _KB]],
                rows.at[slot], g_sems[slot],
            )

        def wait_gather(slot):
            pltpu.make_async_copy(
                emb_hbm.at[src_k.at[0, 0]], rows.at[slot], g_sems[slot]
            ).wait()

        def issue_scatter(t, slot):
            pltpu.async_copy(
                rows.at[slot],
                acc.at[dst_k.at[(t // _KB) % 2, t % _KB]],
                s_sems[slot], add=True,
            )

        def wait_scatter(slot):
            pltpu.make_async_copy(
                rows.at[slot], acc.at[dst_k.at[0, 0]], s_sems[slot]
            ).wait()

        def scale(t, slot):
            bsl = jnp.full((lanes,), (t // _KB) % 2, jnp.int32)
            rr = jnp.full((lanes,), t % _KB, jnp.int32)

            @plsc.parallel_loop(0, b, step=1, unroll=8)
            def _scale_edge(r):
                av = plsc.load_gather(
                    adj_k, [bsl, rr, jnp.full((lanes,), r, jnp.int32)]
                )
                for jj in range(vregs):
                    sl = pl.ds(jj * lanes, lanes)
                    rows[slot, r, sl] = rows[slot, r, sl] * av

        issue_idx(0)
        wait_idx()
        issue_gather(0, 0)

        off = pl.multiple_of(s * rows_per_sub, 8)
        pltpu.sync_copy(
            zeros_hbm.at[pl.ds(off, rows_per_sub)],
            acc.at[pl.ds(off, rows_per_sub)],
        )
        if tail:
            @pl.when(s == 0)
            def _zero_tail():
                pltpu.sync_copy(
                    zeros_hbm.at[pl.ds(tail_start, tail)],
                    acc.at[pl.ds(tail_start, tail)],
                )
        plsc.subcore_barrier()

        nbatch = per_w // _KB
        assert per_w % _NSLOT == 0 and _NSLOT == 5 and per_w % _KB == 0
        inv = pow(_NSLOT, -1, _KB)
        c_issue = [(inv * (1 - j)) % _KB for j in range(_NSLOT)]
        c_wait = [(inv * (_KB - _LEAD - j)) % _KB for j in range(_NSLOT)]
        for pchunk in range(1, _LEAD):
            issue_gather(pchunk, pchunk)

        def gbody(g, carry):
            for j in range(_NSLOT):
                t = g * _NSLOT + j
                tgt = (j + _LEAD) % _NSLOT

                def _do_idx_wait():
                    @pl.when(g % _KB == c_wait[j])
                    def _widx():
                        wait_idx()

                if j < _NSLOT - _LEAD:
                    @pl.when(g > 0)
                    def _drain():
                        wait_scatter(tgt)

                    _do_idx_wait()
                    issue_gather(t + _LEAD, tgt)
                else:
                    wait_scatter(tgt)

                    @pl.when(g < g_iters - 1)
                    def _issue():
                        _do_idx_wait()
                        issue_gather(t + _LEAD, tgt)

                @pl.when(
                    (g % _KB == c_issue[j])
                    & ((g * _NSLOT + j - 1) // _KB + 1 < nbatch)
                )
                def _prefetch_idx():
                    issue_idx((g * _NSLOT + j - 1) // _KB + 1)

                wait_gather(j)
                scale(t, j)
                issue_scatter(t, j)
            return carry

        lax.fori_loop(0, g_iters, gbody, 0)
        for j in range(_NSLOT - _LEAD):
            wait_scatter((j + _LEAD) % _NSLOT)
        plsc.subcore_barrier()

        pltpu.sync_copy(
            acc.at[pl.ds(off, rows_per_sub)],
            part_hbm.at[c, pl.ds(off, rows_per_sub)],
        )
        if tail:
            @pl.when(s == 0)
            def _write_tail():
                pltpu.sync_copy(
                    acc.at[pl.ds(tail_start, tail)],
                    part_hbm.at[c, pl.ds(tail_start, tail)],
                )

    return sc_k(emb, ei3, adj2d, zeros)


def _tc_transform(partials, emb, w1, w2):
    n, d = emb.shape
    nc = partials.shape[0]
    blk = 1000

    def body(p_ref, e_ref, w1_ref, w2_ref, o_ref):
        rel = p_ref[0]
        for i in range(1, nc):
            rel = rel + p_ref[i]
        o_ref[...] = jnp.dot(
            rel, w1_ref[...], preferred_element_type=jnp.float32
        ) + jnp.dot(
            rel * e_ref[...], w2_ref[...], preferred_element_type=jnp.float32
        )

    return pl.pallas_call(
        body,
        grid=(n // blk,),
        in_specs=[
            pl.BlockSpec((nc, blk, d), lambda i: (0, i, 0)),
            pl.BlockSpec((blk, d), lambda i: (i, 0)),
            pl.BlockSpec((d, d), lambda i: (0, 0)),
            pl.BlockSpec((d, d), lambda i: (0, 0)),
        ],
        out_specs=pl.BlockSpec((blk, d), lambda i: (i, 0)),
        out_shape=jax.ShapeDtypeStruct((n, d), jnp.float32),
    )(partials, emb, w1, w2)


def kernel(embeddings, edge_index, adj_values, W1, W2):
    n, d = embeddings.shape
    e = adj_values.shape[0]
    info = plsc.get_sparse_core_info()
    nw = info.num_cores * info.num_subcores

    chunk_unit = _B * nw * _KB
    e_pad = (e + chunk_unit - 1) // chunk_unit * chunk_unit
    pad = e_pad - e
    ei = edge_index
    adj = adj_values
    if pad:
        spread = jnp.arange(pad, dtype=jnp.int32) % jnp.int32(n)
        ei = jnp.concatenate([ei, jnp.broadcast_to(spread, (2, pad))], axis=1)
        adj = jnp.concatenate([adj, jnp.zeros((pad,), jnp.float32)])
    ei3 = ei.reshape(2, e_pad // _B, _B)
    adj2d = adj.reshape(e_pad // _B, _B)
    zeros = jnp.zeros_like(embeddings)
    partials = _sc_segment_sum(embeddings, ei3, adj2d, zeros)
    return _tc_transform(partials, embeddings, W1, W2)


# speedup vs baseline: 1.0178x; 1.0178x over previous
import functools

import jax
import jax.numpy as jnp
from jax import lax
from jax.experimental import pallas as pl
from jax.experimental.pallas import tpu as pltpu
from jax.experimental.pallas import tpu_sc as plsc

_B = 64
_NSLOT = 5
_LEAD = 3
_KB = 8


def _sc_segment_sum(emb, ei3, adj2d, zeros):
    n, d = emb.shape
    _, nrows, b = ei3.shape
    assert b == _B
    info = plsc.get_sparse_core_info()
    nc, ns, lanes = info.num_cores, info.num_subcores, info.num_lanes
    nw = nc * ns
    per_w = nrows // nw
    assert per_w % _KB == 0 and nrows == per_w * nw
    g_iters = per_w // _NSLOT
    vregs = d // lanes

    rows_per_sub = (n // ns) // 8 * 8
    tail_start = ns * rows_per_sub
    tail = n - tail_start

    mesh = plsc.VectorSubcoreMesh(core_axis_name="c", subcore_axis_name="s")

    @functools.partial(
        pl.kernel,
        mesh=mesh,
        out_type=jax.ShapeDtypeStruct((nc, n, d), jnp.float32),
        compiler_params=pltpu.CompilerParams(needs_layout_passes=False),
        scratch_types=[
            pltpu.VMEM((2, _KB, b), jnp.int32),
            pltpu.VMEM((2, _KB, b), jnp.int32),
            pltpu.VMEM((2, _KB, b), jnp.float32),
            pltpu.VMEM((_NSLOT, b, d), jnp.float32),
            pltpu.VMEM_SHARED((n, d), jnp.float32),
            [pltpu.SemaphoreType.DMA] * _NSLOT,
            [pltpu.SemaphoreType.DMA] * _NSLOT,
            pltpu.SemaphoreType.DMA,
        ],
    )
    def sc_k(emb_hbm, ei_hbm, adj_hbm, zeros_hbm, part_hbm,
             src_k, dst_k, adj_k, rows, acc, g_sems, s_sems, i_sem):
        c = lax.axis_index("c")
        s = lax.axis_index("s")
        w = c * ns + s
        start = w * per_w

        def issue_idx(bi):
            off = pl.multiple_of(start + bi * _KB, 8)
            sl = bi % 2
            pltpu.async_copy(ei_hbm.at[1, pl.ds(off, _KB)], src_k.at[sl], i_sem)
            pltpu.async_copy(ei_hbm.at[0, pl.ds(off, _KB)], dst_k.at[sl], i_sem)
            pltpu.async_copy(adj_hbm.at[pl.ds(off, _KB)], adj_k.at[sl], i_sem)

        def wait_idx():
            for ref in (src_k, dst_k, adj_k):
                pltpu.make_async_copy(
                    ei_hbm.at[0, pl.ds(start, _KB)], ref.at[0], i_sem
                ).wait()

        def issue_gather(t, slot):
            pltpu.async_copy(
                emb_hbm.at[src_k.at[(t // _KB) % 2, t % _KB]],
                rows.at[slot], g_sems[slot],
            )

        def wait_gather(slot):
            pltpu.make_async_copy(
                emb_hbm.at[src_k.at[0, 0]], rows.at[slot], g_sems[slot]
            ).wait()

        def issue_scatter(t, slot):
            pltpu.async_copy(
                rows.at[slot],
                acc.at[dst_k.at[(t // _KB) % 2, t % _KB]],
                s_sems[slot], add=True,
            )

        def wait_scatter(slot):
            pltpu.make_async_copy(
                rows.at[slot], acc.at[dst_k.at[0, 0]], s_sems[slot]
            ).wait()

        def scale(t, slot):
            bsl = jnp.full((lanes,), (t // _KB) % 2, jnp.int32)
            rr = jnp.full((lanes,), t % _KB, jnp.int32)

            @plsc.parallel_loop(0, b, step=1, unroll=8)
            def _scale_edge(r):
                av = plsc.load_gather(
                    adj_k, [bsl, rr, jnp.full((lanes,), r, jnp.int32)]
                )
                for jj in range(vregs):
                    sl = pl.ds(jj * lanes, lanes)
                    rows[slot, r, sl] = rows[slot, r, sl] * av

        issue_idx(0)
        wait_idx()
        issue_gather(0, 0)

        off = pl.multiple_of(s * rows_per_sub, 8)
        pltpu.sync_copy(
            zeros_hbm.at[pl.ds(off, rows_per_sub)],
            acc.at[pl.ds(off, rows_per_sub)],
        )
        if tail:
            @pl.when(s == 0)
            def _zero_tail():
                pltpu.sync_copy(
                    zeros_hbm.at[pl.ds(tail_start, tail)],
                    acc.at[pl.ds(tail_start, tail)],
                )
        plsc.subcore_barrier()

        nbatch = per_w // _KB
        assert per_w % _NSLOT == 0 and _NSLOT == 5 and per_w % _KB == 0
        inv = pow(_NSLOT, -1, _KB)
        c_issue = [(inv * (1 - j)) % _KB for j in range(_NSLOT)]
        c_wait = [(inv * (_KB - _LEAD - j)) % _KB for j in range(_NSLOT)]
        for pchunk in range(1, _LEAD):
            issue_gather(pchunk, pchunk)

        def gbody(g, carry):
            for j in range(_NSLOT):
                t = g * _NSLOT + j
                tgt = (j + _LEAD) % _NSLOT

                def _do_idx_wait():
                    @pl.when(g % _KB == c_wait[j])
                    def _widx():
                        wait_idx()

                if j < _NSLOT - _LEAD:
                    @pl.when(g > 0)
                    def _drain():
                        wait_scatter(tgt)

                    _do_idx_wait()
                    issue_gather(t + _LEAD, tgt)
                else:
                    wait_scatter(tgt)

                    @pl.when(g < g_iters - 1)
                    def _issue():
                        _do_idx_wait()
                        issue_gather(t + _LEAD, tgt)

                @pl.when(
                    (g % _KB == c_issue[j])
                    & ((g * _NSLOT + j - 1) // _KB + 1 < nbatch)
                )
                def _prefetch_idx():
                    issue_idx((g * _NSLOT + j - 1) // _KB + 1)

                wait_gather(j)
                scale(t, j)
                issue_scatter(t, j)
            return carry

        lax.fori_loop(0, g_iters, gbody, 0)
        for j in range(_NSLOT - _LEAD):
            wait_scatter((j + _LEAD) % _NSLOT)
        plsc.subcore_barrier()

        pltpu.sync_copy(
            acc.at[pl.ds(off, rows_per_sub)],
            part_hbm.at[c, pl.ds(off, rows_per_sub)],
        )
        if tail:
            @pl.when(s == 0)
            def _write_tail():
                pltpu.sync_copy(
                    acc.at[pl.ds(tail_start, tail)],
                    part_hbm.at[c, pl.ds(tail_start, tail)],
                )

    return sc_k(emb, ei3, adj2d, zeros)


def _tc_transform(partials, emb, w1, w2):
    n, d = emb.shape
    nc = partials.shape[0]
    blk = 2000

    def body(p_ref, e_ref, w1_ref, w2_ref, o_ref):
        rel = p_ref[0]
        for i in range(1, nc):
            rel = rel + p_ref[i]
        o_ref[...] = jnp.dot(
            rel, w1_ref[...], preferred_element_type=jnp.float32
        ) + jnp.dot(
            rel * e_ref[...], w2_ref[...], preferred_element_type=jnp.float32
        )

    return pl.pallas_call(
        body,
        grid=(n // blk,),
        in_specs=[
            pl.BlockSpec((nc, blk, d), lambda i: (0, i, 0)),
            pl.BlockSpec((blk, d), lambda i: (i, 0)),
            pl.BlockSpec((d, d), lambda i: (0, 0)),
            pl.BlockSpec((d, d), lambda i: (0, 0)),
        ],
        out_specs=pl.BlockSpec((blk, d), lambda i: (i, 0)),
        out_shape=jax.ShapeDtypeStruct((n, d), jnp.float32),
    )(partials, emb, w1, w2)


def kernel(embeddings, edge_index, adj_values, W1, W2):
    n, d = embeddings.shape
    e = adj_values.shape[0]
    info = plsc.get_sparse_core_info()
    nw = info.num_cores * info.num_subcores

    chunk_unit = _B * nw * _KB
    e_pad = (e + chunk_unit - 1) // chunk_unit * chunk_unit
    pad = e_pad - e
    ei = edge_index
    adj = adj_values
    if pad:
        spread = jnp.arange(pad, dtype=jnp.int32) % jnp.int32(n)
        ei = jnp.concatenate([ei, jnp.broadcast_to(spread, (2, pad))], axis=1)
        adj = jnp.concatenate([adj, jnp.zeros((pad,), jnp.float32)])
    ei3 = ei.reshape(2, e_pad // _B, _B)
    adj2d = adj.reshape(e_pad // _B, _B)
    zeros = jnp.zeros_like(embeddings)
    partials = _sc_segment_sum(embeddings, ei3, adj2d, zeros)
    return _tc_transform(partials, embeddings, W1, W2)


# speedup vs baseline: 1.0248x; 1.0069x over previous
import functools

import jax
import jax.numpy as jnp
from jax import lax
from jax.experimental import pallas as pl
from jax.experimental.pallas import tpu as pltpu
from jax.experimental.pallas import tpu_sc as plsc

_B = 64
_NSLOT = 5
_LEAD = 3
_KB = 8


def _sc_segment_sum(emb, ei3, adj2d, zeros):
    n, d = emb.shape
    _, nrows, b = ei3.shape
    assert b == _B
    info = plsc.get_sparse_core_info()
    nc, ns, lanes = info.num_cores, info.num_subcores, info.num_lanes
    nw = nc * ns
    per_w = nrows // nw
    assert per_w % _KB == 0 and nrows == per_w * nw
    g_iters = per_w // _NSLOT
    vregs = d // lanes

    rows_per_sub = (n // ns) // 8 * 8
    tail_start = ns * rows_per_sub
    tail = n - tail_start

    mesh = plsc.VectorSubcoreMesh(core_axis_name="c", subcore_axis_name="s")

    @functools.partial(
        pl.kernel,
        mesh=mesh,
        out_type=jax.ShapeDtypeStruct((nc, n, d), jnp.float32),
        compiler_params=pltpu.CompilerParams(needs_layout_passes=False),
        scratch_types=[
            pltpu.VMEM((2, _KB, b), jnp.int32),
            pltpu.VMEM((2, _KB, b), jnp.int32),
            pltpu.VMEM((2, _KB, b), jnp.float32),
            pltpu.VMEM((_NSLOT, b, d), jnp.float32),
            pltpu.VMEM_SHARED((n, d), jnp.float32),
            [pltpu.SemaphoreType.DMA] * _NSLOT,
            [pltpu.SemaphoreType.DMA] * _NSLOT,
            pltpu.SemaphoreType.DMA,
        ],
    )
    def sc_k(emb_hbm, ei_hbm, adj_hbm, zeros_hbm, part_hbm,
             src_k, dst_k, adj_k, rows, acc, g_sems, s_sems, i_sem):
        c = lax.axis_index("c")
        s = lax.axis_index("s")
        w = c * ns + s
        start = w * per_w

        def issue_idx(bi):
            off = pl.multiple_of(start + bi * _KB, 8)
            sl = bi % 2
            pltpu.async_copy(ei_hbm.at[1, pl.ds(off, _KB)], src_k.at[sl], i_sem)
            pltpu.async_copy(ei_hbm.at[0, pl.ds(off, _KB)], dst_k.at[sl], i_sem)
            pltpu.async_copy(adj_hbm.at[pl.ds(off, _KB)], adj_k.at[sl], i_sem)

        def wait_idx():
            for ref in (src_k, dst_k, adj_k):
                pltpu.make_async_copy(
                    ei_hbm.at[0, pl.ds(start, _KB)], ref.at[0], i_sem
                ).wait()

        def issue_gather(t, slot):
            pltpu.async_copy(
                emb_hbm.at[src_k.at[(t // _KB) % 2, t % _KB]],
                rows.at[slot], g_sems[slot],
            )

        def wait_gather(slot):
            pltpu.make_async_copy(
                emb_hbm.at[src_k.at[0, 0]], rows.at[slot], g_sems[slot]
            ).wait()

        def issue_scatter(t, slot):
            pltpu.async_copy(
                rows.at[slot],
                acc.at[dst_k.at[(t // _KB) % 2, t % _KB]],
                s_sems[slot], add=True,
            )

        def wait_scatter(slot):
            pltpu.make_async_copy(
                rows.at[slot], acc.at[dst_k.at[0, 0]], s_sems[slot]
            ).wait()

        def scale(t, slot):
            bsl = jnp.full((lanes,), (t // _KB) % 2, jnp.int32)
            rr = jnp.full((lanes,), t % _KB, jnp.int32)

            @plsc.parallel_loop(0, b, step=1, unroll=4)
            def _scale_edge(r):
                av = plsc.load_gather(
                    adj_k, [bsl, rr, jnp.full((lanes,), r, jnp.int32)]
                )
                for jj in range(vregs):
                    sl = pl.ds(jj * lanes, lanes)
                    rows[slot, r, sl] = rows[slot, r, sl] * av

        issue_idx(0)
        wait_idx()
        issue_gather(0, 0)

        off = pl.multiple_of(s * rows_per_sub, 8)
        pltpu.sync_copy(
            zeros_hbm.at[pl.ds(off, rows_per_sub)],
            acc.at[pl.ds(off, rows_per_sub)],
        )
        if tail:
            @pl.when(s == 0)
            def _zero_tail():
                pltpu.sync_copy(
                    zeros_hbm.at[pl.ds(tail_start, tail)],
                    acc.at[pl.ds(tail_start, tail)],
                )
        plsc.subcore_barrier()

        nbatch = per_w // _KB
        assert per_w % _NSLOT == 0 and _NSLOT == 5 and per_w % _KB == 0
        inv = pow(_NSLOT, -1, _KB)
        c_issue = [(inv * (1 - j)) % _KB for j in range(_NSLOT)]
        c_wait = [(inv * (_KB - _LEAD - j)) % _KB for j in range(_NSLOT)]
        for pchunk in range(1, _LEAD):
            issue_gather(pchunk, pchunk)

        def gbody(g, carry):
            for j in range(_NSLOT):
                t = g * _NSLOT + j
                tgt = (j + _LEAD) % _NSLOT

                def _do_idx_wait():
                    @pl.when(g % _KB == c_wait[j])
                    def _widx():
                        wait_idx()

                if j < _NSLOT - _LEAD:
                    @pl.when(g > 0)
                    def _drain():
                        wait_scatter(tgt)

                    _do_idx_wait()
                    issue_gather(t + _LEAD, tgt)
                else:
                    wait_scatter(tgt)

                    @pl.when(g < g_iters - 1)
                    def _issue():
                        _do_idx_wait()
                        issue_gather(t + _LEAD, tgt)

                @pl.when(
                    (g % _KB == c_issue[j])
                    & ((g * _NSLOT + j - 1) // _KB + 1 < nbatch)
                )
                def _prefetch_idx():
                    issue_idx((g * _NSLOT + j - 1) // _KB + 1)

                wait_gather(j)
                scale(t, j)
                issue_scatter(t, j)
            return carry

        lax.fori_loop(0, g_iters, gbody, 0)
        for j in range(_NSLOT - _LEAD):
            wait_scatter((j + _LEAD) % _NSLOT)
        plsc.subcore_barrier()

        pltpu.sync_copy(
            acc.at[pl.ds(off, rows_per_sub)],
            part_hbm.at[c, pl.ds(off, rows_per_sub)],
        )
        if tail:
            @pl.when(s == 0)
            def _write_tail():
                pltpu.sync_copy(
                    acc.at[pl.ds(tail_start, tail)],
                    part_hbm.at[c, pl.ds(tail_start, tail)],
                )

    return sc_k(emb, ei3, adj2d, zeros)


def _tc_transform(partials, emb, w1, w2):
    n, d = emb.shape
    nc = partials.shape[0]
    blk = 2000

    def body(p_ref, e_ref, w1_ref, w2_ref, o_ref):
        rel = p_ref[0]
        for i in range(1, nc):
            rel = rel + p_ref[i]
        o_ref[...] = jnp.dot(
            rel, w1_ref[...], preferred_element_type=jnp.float32
        ) + jnp.dot(
            rel * e_ref[...], w2_ref[...], preferred_element_type=jnp.float32
        )

    return pl.pallas_call(
        body,
        grid=(n // blk,),
        in_specs=[
            pl.BlockSpec((nc, blk, d), lambda i: (0, i, 0)),
            pl.BlockSpec((blk, d), lambda i: (i, 0)),
            pl.BlockSpec((d, d), lambda i: (0, 0)),
            pl.BlockSpec((d, d), lambda i: (0, 0)),
        ],
        out_specs=pl.BlockSpec((blk, d), lambda i: (i, 0)),
        out_shape=jax.ShapeDtypeStruct((n, d), jnp.float32),
    )(partials, emb, w1, w2)


def kernel(embeddings, edge_index, adj_values, W1, W2):
    n, d = embeddings.shape
    e = adj_values.shape[0]
    info = plsc.get_sparse_core_info()
    nw = info.num_cores * info.num_subcores

    chunk_unit = _B * nw * _KB
    e_pad = (e + chunk_unit - 1) // chunk_unit * chunk_unit
    pad = e_pad - e
    ei = edge_index
    adj = adj_values
    if pad:
        spread = jnp.arange(pad, dtype=jnp.int32) % jnp.int32(n)
        ei = jnp.concatenate([ei, jnp.broadcast_to(spread, (2, pad))], axis=1)
        adj = jnp.concatenate([adj, jnp.zeros((pad,), jnp.float32)])
    ei3 = ei.reshape(2, e_pad // _B, _B)
    adj2d = adj.reshape(e_pad // _B, _B)
    zeros = jnp.zeros_like(embeddings)
    partials = _sc_segment_sum(embeddings, ei3, adj2d, zeros)
    return _tc_transform(partials, embeddings, W1, W2)
